# baseline (speedup 1.0000x reference)
import functools
import math

import jax
import jax.numpy as jnp
from jax import lax
from jax.experimental import pallas as pl
from jax.experimental.pallas import tpu as pltpu
from jax.experimental.pallas import tpu_sc as plsc

NC = 2
NS = 16
NW = NC * NS
LANES = 16

GATHER_ROWS = 128
TCOLS = 128
TBATCH = 8


def _make_pe(seq_len: int, d: int) -> jax.Array:
    pos = jnp.arange(0, seq_len, dtype=jnp.float32)[:, None]
    fill = pos * jnp.exp(
        -jnp.arange(0, d, 2, dtype=jnp.float32) * math.log(10000.0) / d
    )
    pe = jnp.zeros((seq_len, d), dtype=jnp.float32)
    pe = pe.at[:, 0::2].set(jnp.sin(fill))
    pe = pe.at[:, 1::2].set(jnp.cos(fill))
    return pe


@functools.partial(jax.jit, static_argnames=("v", "d"))
def _sc_pack(tt, tail2, *, v, d):
    n_tile_cols = v // TCOLS
    tail_cols = v - n_tile_cols * TCOLS
    n_batches = n_tile_cols // TBATCH
    rem_tiles = n_tile_cols - n_batches * TBATCH
    per_w = (n_batches + NW - 1) // NW
    pairs_per_col = TCOLS // 2
    sw = TBATCH * TCOLS

    mesh = plsc.VectorSubcoreMesh(core_axis_name="c", subcore_axis_name="s")

    @functools.partial(
        pl.kernel,
        out_type=jax.ShapeDtypeStruct((v // 2, 2 * d), jnp.float32),
        mesh=mesh,
        compiler_params=pltpu.CompilerParams(
            use_tc_tiling_on_sc=True, needs_layout_passes=False
        ),
        scratch_types=[
            pltpu.VMEM((d, sw), jnp.float32),
            pltpu.VMEM((TBATCH * pairs_per_col, 2 * d), jnp.float32),
            pltpu.SemaphoreType.DMA,
        ],
    )
    def body(tt_hbm, tail2_hbm, out_hbm, stage_v, pack_v, sem):
        wid = lax.axis_index("s") * NC + lax.axis_index("c")

        lane = lax.iota(jnp.int32, LANES)

        def pack_cols(n_cols_tiles):
            @plsc.parallel_loop(0, n_cols_tiles * pairs_per_col, unroll=8)
            def pair_body(q):
                for h in range(2):
                    c = jnp.full((LANES,), 2 * q + h, jnp.int32)
                    for j in range(d // LANES):
                        val = plsc.load_gather(
                            stage_v, [j * LANES + lane, c]
                        )
                        pack_v[q, pl.ds(h * d + j * LANES, LANES)] = val

        def do_batch(b0, n_cols_tiles):
            w = n_cols_tiles * TCOLS
            pltpu.sync_copy(
                tt_hbm.at[:, pl.ds(b0 * TCOLS, w)],
                stage_v.at[:, pl.ds(0, w)],
            )
            pack_cols(n_cols_tiles)
            pltpu.sync_copy(
                pack_v.at[pl.ds(0, n_cols_tiles * pairs_per_col)],
                out_hbm.at[
                    pl.ds(b0 * pairs_per_col, n_cols_tiles * pairs_per_col)
                ],
            )

        def batch_body(k, carry):
            bid = wid + k * NW

            @pl.when(bid < n_batches)
            def _():
                do_batch(bid * TBATCH, TBATCH)

            return carry

        lax.fori_loop(0, per_w, batch_body, 0)

        @pl.when(wid == 0)
        def _():
            if rem_tiles > 0:
                do_batch(n_batches * TBATCH, rem_tiles)
            if tail_cols > 0:
                pltpu.sync_copy(tail2_hbm, pack_v.at[pl.ds(0, tail_cols // 2)])
                pltpu.sync_copy(
                    pack_v.at[pl.ds(0, tail_cols // 2)],
                    out_hbm.at[
                        pl.ds(n_tile_cols * pairs_per_col, tail_cols // 2)
                    ],
                )

    return body(tt, tail2)


@functools.partial(jax.jit, static_argnames=("n_rows", "d", "seq_len"))
def _sc_embed(x2d, pe, table2, *, n_rows, d, seq_len):
    per_w = n_rows // NW
    n_g = per_w // GATHER_ROWS
    chunk = 256
    n_chunks = per_w // chunk
    g_per_chunk = chunk // GATHER_ROWS
    vecs_per_row = d // LANES

    mesh = plsc.VectorSubcoreMesh(core_axis_name="c", subcore_axis_name="s")

    @functools.partial(
        pl.kernel,
        out_type=jax.ShapeDtypeStruct((n_rows, d), jnp.float32),
        mesh=mesh,
        compiler_params=pltpu.CompilerParams(use_tc_tiling_on_sc=True),
        scratch_types=[
            pltpu.VMEM((n_g, GATHER_ROWS), jnp.int32),
            pltpu.VMEM((n_g, GATHER_ROWS), jnp.int32),
            pltpu.VMEM((per_w // LANES, LANES), jnp.float32),
            pltpu.VMEM((chunk, 2 * d), jnp.float32),
            pltpu.VMEM((chunk, d), jnp.float32),
            pltpu.SemaphoreType.DMA,
        ],
    )
    def body(x_hbm, pe_hbm, table2_hbm, out_hbm,
             idx_v, pidx_v, par_v, pairs_v, out_v, sem):
        wid = lax.axis_index("s") * NC + lax.axis_index("c")
        base = wid * per_w
        l_start = lax.rem(base, seq_len)

        pltpu.sync_copy(x_hbm.at[pl.ds(wid * n_g, n_g)], idx_v)

        for u in range(per_w // LANES):
            r = u // (GATHER_ROWS // LANES)
            c0 = (u % (GATHER_ROWS // LANES)) * LANES
            tok = idx_v[r, pl.ds(c0, LANES)]
            pidx_v[r, pl.ds(c0, LANES)] = lax.shift_right_logical(tok, 1)
            par_v[u, :] = lax.convert_element_type(
                lax.bitwise_and(tok, 1), jnp.float32
            )

        for c in range(n_chunks):
            copies = [
                pltpu.async_copy(
                    table2_hbm.at[pidx_v.at[c * g_per_chunk + k]],
                    pairs_v.at[pl.ds(k * GATHER_ROWS, GATHER_ROWS)],
                    sem,
                )
                for k in range(g_per_chunk)
            ]
            pltpu.sync_copy(
                pe_hbm.at[pl.ds(l_start + c * chunk, chunk)], out_v
            )
            for cp in copies:
                cp.wait()

            def row_fix(i, carry):
                t = c * chunk + i
                g = lax.div(t, LANES)
                ln = lax.rem(t, LANES)
                par_vec = par_v[g, :]
                pf = par_vec[jnp.full((LANES,), ln, jnp.int32)]
                for j in range(vecs_per_row):
                    lo = pairs_v[i, pl.ds(j * LANES, LANES)]
                    hi = pairs_v[i, pl.ds(d + j * LANES, LANES)]
                    val = lo + pf * (hi - lo)
                    plsc.addupdate(out_v.at[i, pl.ds(j * LANES, LANES)], val)
                return carry

            lax.fori_loop(0, chunk, row_fix, 0)
            pltpu.sync_copy(out_v, out_hbm.at[pl.ds(base + c * chunk, chunk)])

    return body(x2d, pe, table2)


def kernel(x, table):
    b, l = x.shape
    v, d = table.shape
    n_rows = b * l
    pe = _make_pe(l, d)
    x2d = x.reshape(n_rows // GATHER_ROWS, GATHER_ROWS).astype(jnp.int32)
    n_full = (v // TCOLS) * TCOLS
    tail2 = table[n_full:, :].reshape((v - n_full) // 2, 2 * d)
    table2 = _sc_pack(table.T, tail2, v=v, d=d)
    out = _sc_embed(x2d, pe, table2, n_rows=n_rows, d=d, seq_len=l)
    return out.reshape(b, l, d)


# speedup vs baseline: 1.6347x; 1.6347x over previous
import functools
import math

import jax
import jax.numpy as jnp
from jax import lax
from jax.experimental import pallas as pl
from jax.experimental.pallas import tpu as pltpu
from jax.experimental.pallas import tpu_sc as plsc

NC = 2
NS = 16
NW = NC * NS
LANES = 16

GATHER_ROWS = 128


def _make_pe(seq_len: int, d: int) -> jax.Array:
    pos = jnp.arange(0, seq_len, dtype=jnp.float32)[:, None]
    fill = pos * jnp.exp(
        -jnp.arange(0, d, 2, dtype=jnp.float32) * math.log(10000.0) / d
    )
    pe = jnp.zeros((seq_len, d), dtype=jnp.float32)
    pe = pe.at[:, 0::2].set(jnp.sin(fill))
    pe = pe.at[:, 1::2].set(jnp.cos(fill))
    return pe


@functools.partial(jax.jit, static_argnames=("n_rows", "d", "seq_len"))
def _sc_embed(x2d, pe, table, *, n_rows, d, seq_len):
    per_w = n_rows // NW
    n_g = per_w // GATHER_ROWS
    half = per_w // 2
    g_per_half = n_g // 2
    vecs_per_row = d // LANES

    mesh = plsc.VectorSubcoreMesh(core_axis_name="c", subcore_axis_name="s")

    @functools.partial(
        pl.kernel,
        out_type=jax.ShapeDtypeStruct((n_rows, d), jnp.float32),
        mesh=mesh,
        compiler_params=pltpu.CompilerParams(
            use_tc_tiling_on_sc=False, needs_layout_passes=False
        ),
        scratch_types=[
            pltpu.VMEM((n_g, GATHER_ROWS), jnp.int32),
            pltpu.VMEM((half, d), jnp.float32),
            pltpu.VMEM((half, d), jnp.float32),
            pltpu.SemaphoreType.DMA,
        ],
    )
    def body(x_hbm, pe_hbm, table_hbm, out_hbm, idx_v, rows_v, pe_v, sem):
        wid = lax.axis_index("s") * NC + lax.axis_index("c")
        base = wid * per_w
        l_start = lax.rem(base, seq_len)

        pltpu.sync_copy(x_hbm.at[pl.ds(wid * n_g, n_g)], idx_v)

        for c in range(2):
            copies = [
                pltpu.async_copy(
                    table_hbm.at[idx_v.at[c * g_per_half + k]],
                    rows_v.at[pl.ds(k * GATHER_ROWS, GATHER_ROWS)],
                    sem,
                )
                for k in range(g_per_half)
            ]
            pltpu.sync_copy(pe_hbm.at[pl.ds(l_start + c * half, half)], pe_v)
            for cp in copies:
                cp.wait()

            def row_add(i, carry):
                for j in range(vecs_per_row):
                    sl = pl.ds(j * LANES, LANES)
                    plsc.addupdate(rows_v.at[i, sl], pe_v[i, sl])
                return carry

            lax.fori_loop(0, half, row_add, 0)
            pltpu.sync_copy(rows_v, out_hbm.at[pl.ds(base + c * half, half)])

    return body(x2d, pe, table)


def kernel(x, table):
    b, l = x.shape
    v, d = table.shape
    n_rows = b * l
    pe = _make_pe(l, d)
    x2d = x.reshape(n_rows // GATHER_ROWS, GATHER_ROWS).astype(jnp.int32)
    out = _sc_embed(x2d, pe, table, n_rows=n_rows, d=d, seq_len=l)
    return out.reshape(b, l, d)
